# baseline (device time: 25805 ns/iter reference)
import os

import jax
import jax.numpy as jnp
from jax import lax
from jax.experimental import pallas as pl
from jax.experimental.pallas import tpu as pltpu

_PHASES = int(os.environ.get("KERNEL_PHASES", "2"))

N_DEV = 16
N_TOK = 512
D_IN = 256
D_OUT = 512
E_LOCAL = 4
N_EXP = 64
ROWS = N_TOK // N_DEV
N_GROUP = 2
G_TOK = N_TOK // N_GROUP
G_CHUNKS = N_DEV // N_GROUP


def kernel(x, router_W, route_idx, expert_W, shared_W):
    def body(
        x_ref,
        rw_ref,
        idx_ref,
        ew_ref,
        sw_ref,
        out_ref,
        acc_ref,
        rs_buf,
        rs_ssem,
        rs_rsem,
        ag_ssem,
        ag_rsem,
    ):
        my = lax.axis_index("i")

        if _PHASES > 0:
            barrier_sem = pltpu.get_barrier_semaphore()
            for d in range(1, N_DEV):
                pl.semaphore_signal(
                    barrier_sem,
                    inc=1,
                    device_id=((my + d) % N_DEV,),
                    device_id_type=pl.DeviceIdType.MESH,
                )

        xb = x_ref[...].astype(jnp.bfloat16)
        scores = jnp.dot(
            xb, rw_ref[...].astype(jnp.bfloat16), preferred_element_type=jnp.float32
        )
        s_max = jnp.max(scores, axis=-1, keepdims=True)
        e_s = jnp.exp(scores - s_max)
        probs = e_s / jnp.sum(e_s, axis=-1, keepdims=True)
        idx = idx_ref[...]
        cols = lax.broadcasted_iota(jnp.int32, (N_TOK, N_EXP), 1)
        p_chosen = jnp.sum(
            jnp.where(cols == idx, probs, 0.0), axis=-1, keepdims=True
        )

        wm = ew_ref[...].astype(jnp.bfloat16).reshape(E_LOCAL * D_IN, D_OUT)

        def rs_copy(d):
            return pltpu.make_async_remote_copy(
                src_ref=acc_ref.at[(my + d) % N_DEV],
                dst_ref=rs_buf.at[d - 1],
                send_sem=rs_ssem.at[d - 1],
                recv_sem=rs_rsem.at[d - 1],
                device_id=((my + d) % N_DEV,),
                device_id_type=pl.DeviceIdType.MESH,
            )

        for g in range(N_GROUP):
            rsl = slice(g * G_TOK, (g + 1) * G_TOK)
            xw_g = jnp.concatenate(
                [
                    xb[rsl]
                    * jnp.where(
                        idx[rsl] == my * E_LOCAL + e, p_chosen[rsl], 0.0
                    ).astype(jnp.bfloat16)
                    for e in range(E_LOCAL)
                ],
                axis=1,
            )
            partial_g = jnp.dot(xw_g, wm, preferred_element_type=jnp.float32)
            acc_ref[g * G_CHUNKS : (g + 1) * G_CHUNKS] = partial_g.reshape(
                G_CHUNKS, ROWS, D_OUT
            ).astype(jnp.bfloat16)

            if _PHASES == 0:
                continue
            if g == 0:
                pl.semaphore_wait(barrier_sem, N_DEV - 1)
            for d in range(1, N_DEV):
                in_group = ((my + d) % N_DEV) // G_CHUNKS == g

                @pl.when(in_group)
                def _():
                    rs_copy(d).start()

        if _PHASES == 0:
            out_ref[...] = acc_ref[...].reshape(N_TOK, D_OUT)
            return

        shared_own = jnp.dot(
            x_ref[pl.ds(my * ROWS, ROWS), :].astype(jnp.bfloat16),
            sw_ref[...].astype(jnp.bfloat16),
            preferred_element_type=jnp.float32,
        )

        rs = [rs_copy(d) for d in range(1, N_DEV)]
        for c in rs:
            c.wait_send()
        for c in rs:
            c.wait_recv()

        own = acc_ref[my].astype(jnp.float32)
        red = own + jnp.sum(rs_buf[...].astype(jnp.float32), axis=0) + shared_own
        out_ref[pl.ds(my * ROWS, ROWS), :] = red.astype(jnp.bfloat16)

        if _PHASES == 1:
            return

        ag = []
        for d in range(1, N_DEV):
            c = pltpu.make_async_remote_copy(
                src_ref=out_ref.at[pl.ds(my * ROWS, ROWS)],
                dst_ref=out_ref.at[pl.ds(my * ROWS, ROWS)],
                send_sem=ag_ssem.at[d - 1],
                recv_sem=ag_rsem.at[d - 1],
                device_id=((my + d) % N_DEV,),
                device_id_type=pl.DeviceIdType.MESH,
            )
            c.start()
            ag.append(c)
        for c in ag:
            c.wait_send()
        for c in ag:
            c.wait_recv()

    return pl.pallas_call(
        body,
        out_shape=jax.ShapeDtypeStruct((N_TOK, D_OUT), jnp.bfloat16),
        in_specs=[pl.BlockSpec(memory_space=pltpu.VMEM)] * 5,
        out_specs=pl.BlockSpec(memory_space=pltpu.VMEM),
        scratch_shapes=[
            pltpu.VMEM((N_DEV, ROWS, D_OUT), jnp.bfloat16),
            pltpu.VMEM((N_DEV - 1, ROWS, D_OUT), jnp.bfloat16),
            pltpu.SemaphoreType.DMA((N_DEV - 1,)),
            pltpu.SemaphoreType.DMA((N_DEV - 1,)),
            pltpu.SemaphoreType.DMA((N_DEV - 1,)),
            pltpu.SemaphoreType.DMA((N_DEV - 1,)),
        ],
        **(
            {"compiler_params": pltpu.CompilerParams(collective_id=0)}
            if _PHASES > 0
            else {}
        ),
    )(x, router_W, route_idx, expert_W, shared_W)


# device time: 25584 ns/iter; 1.0086x vs baseline; 1.0086x over previous
import os

import jax
import jax.numpy as jnp
from jax import lax
from jax.experimental import pallas as pl
from jax.experimental.pallas import tpu as pltpu

_PHASES = int(os.environ.get("KERNEL_PHASES", "2"))

N_DEV = 16
N_TOK = 512
D_IN = 256
D_OUT = 512
E_LOCAL = 4
N_EXP = 64
ROWS = N_TOK // N_DEV
N_HALF = 2
COLS = D_OUT // N_HALF
N_PEER = N_DEV - 1


def kernel(x, router_W, route_idx, expert_W, shared_W):
    def body(
        x_ref,
        rw_ref,
        idx_ref,
        ew_ref,
        sw_ref,
        out_ref,
        acc_ref,
        rs_buf,
        rs_ssem,
        rs_rsem,
        ag_ssem,
        ag_rsem,
    ):
        my = lax.axis_index("i")

        if _PHASES > 0:
            barrier_sem = pltpu.get_barrier_semaphore()
            for d in range(1, N_DEV):
                pl.semaphore_signal(
                    barrier_sem,
                    inc=1,
                    device_id=((my + d) % N_DEV,),
                    device_id_type=pl.DeviceIdType.MESH,
                )

        xb = x_ref[...].astype(jnp.bfloat16)
        scores = jnp.dot(
            xb, rw_ref[...].astype(jnp.bfloat16), preferred_element_type=jnp.float32
        )
        s_max = jnp.max(scores, axis=-1, keepdims=True)
        e_s = jnp.exp(scores - s_max)
        probs = e_s / jnp.sum(e_s, axis=-1, keepdims=True)
        idx = idx_ref[...]
        cols = lax.broadcasted_iota(jnp.int32, (N_TOK, N_EXP), 1)
        p_chosen = jnp.sum(
            jnp.where(cols == idx, probs, 0.0), axis=-1, keepdims=True
        )

        xw = jnp.concatenate(
            [
                xb
                * jnp.where(idx == my * E_LOCAL + e, p_chosen, 0.0).astype(
                    jnp.bfloat16
                )
                for e in range(E_LOCAL)
            ],
            axis=1,
        )
        wm = ew_ref[...].astype(jnp.bfloat16).reshape(E_LOCAL * D_IN, D_OUT)

        def rs_copy(h, d):
            return pltpu.make_async_remote_copy(
                src_ref=acc_ref.at[h * N_DEV + (my + d) % N_DEV],
                dst_ref=rs_buf.at[h * N_PEER + d - 1],
                send_sem=rs_ssem.at[h * N_PEER + d - 1],
                recv_sem=rs_rsem.at[h * N_PEER + d - 1],
                device_id=((my + d) % N_DEV,),
                device_id_type=pl.DeviceIdType.MESH,
            )

        for h in range(N_HALF):
            partial_h = jnp.dot(
                xw, wm[:, h * COLS : (h + 1) * COLS],
                preferred_element_type=jnp.float32,
            )
            acc_ref[h * N_DEV : (h + 1) * N_DEV] = partial_h.reshape(
                N_DEV, ROWS, COLS
            ).astype(jnp.bfloat16)

            if _PHASES == 0:
                continue
            if h == 0:
                pl.semaphore_wait(barrier_sem, N_DEV - 1)
            for d in range(1, N_DEV):
                rs_copy(h, d).start()

        if _PHASES == 0:
            out_ref[...] = jnp.concatenate(
                [
                    acc_ref[h * N_DEV : (h + 1) * N_DEV].reshape(N_TOK, COLS)
                    for h in range(N_HALF)
                ],
                axis=1,
            )
            return

        shared_own = jnp.dot(
            x_ref[pl.ds(my * ROWS, ROWS), :].astype(jnp.bfloat16),
            sw_ref[...].astype(jnp.bfloat16),
            preferred_element_type=jnp.float32,
        )

        ag = []
        for h in range(N_HALF):
            rs = [rs_copy(h, d) for d in range(1, N_DEV)]
            for c in rs:
                c.wait_send()
            for c in rs:
                c.wait_recv()

            own = acc_ref[my + h * N_DEV].astype(jnp.float32)
            red = (
                own
                + jnp.sum(
                    rs_buf[h * N_PEER : (h + 1) * N_PEER].astype(jnp.float32),
                    axis=0,
                )
                + shared_own[:, h * COLS : (h + 1) * COLS]
            )
            out_ref[pl.ds(my * ROWS, ROWS), h * COLS : (h + 1) * COLS] = red.astype(
                jnp.bfloat16
            )

            if _PHASES == 1:
                continue
            for d in range(1, N_DEV):
                c = pltpu.make_async_remote_copy(
                    src_ref=out_ref.at[
                        pl.ds(my * ROWS, ROWS), h * COLS : (h + 1) * COLS
                    ],
                    dst_ref=out_ref.at[
                        pl.ds(my * ROWS, ROWS), h * COLS : (h + 1) * COLS
                    ],
                    send_sem=ag_ssem.at[h * N_PEER + d - 1],
                    recv_sem=ag_rsem.at[h * N_PEER + d - 1],
                    device_id=((my + d) % N_DEV,),
                    device_id_type=pl.DeviceIdType.MESH,
                )
                c.start()
                ag.append(c)

        for c in ag:
            c.wait_send()
        for c in ag:
            c.wait_recv()

    return pl.pallas_call(
        body,
        out_shape=jax.ShapeDtypeStruct((N_TOK, D_OUT), jnp.bfloat16),
        in_specs=[pl.BlockSpec(memory_space=pltpu.VMEM)] * 5,
        out_specs=pl.BlockSpec(memory_space=pltpu.VMEM),
        scratch_shapes=[
            pltpu.VMEM((N_HALF * N_DEV, ROWS, COLS), jnp.bfloat16),
            pltpu.VMEM((N_HALF * N_PEER, ROWS, COLS), jnp.bfloat16),
            pltpu.SemaphoreType.DMA((N_HALF * N_PEER,)),
            pltpu.SemaphoreType.DMA((N_HALF * N_PEER,)),
            pltpu.SemaphoreType.DMA((N_HALF * N_PEER,)),
            pltpu.SemaphoreType.DMA((N_HALF * N_PEER,)),
        ],
        **(
            {"compiler_params": pltpu.CompilerParams(collective_id=0)}
            if _PHASES > 0
            else {}
        ),
    )(x, router_W, route_idx, expert_W, shared_W)
